# Initial kernel scaffold; baseline (speedup 1.0000x reference)
import jax
import jax.numpy as jnp
from jax.experimental import pallas as pl


def kernel(x, pos, batch, edge_index, padded_node_mask, padded_edge_mask, eW_in, eb_in, eW_h1, eb_h1, eW_h2, eb_h2, eW_out, eb_out, e_rbf_W, e_out_W, fW_in, fb_in, fW_h1, fb_h1, fW_h2, fb_h2, fW_out, fb_out, f_rbf_W, f_out_W):
    raise NotImplementedError("write your pallas kernel here")


# speedup vs baseline: 2.5199x; 2.5199x over previous
import functools

import jax
import jax.numpy as jnp
from jax import lax
from jax.experimental import pallas as pl
from jax.experimental.pallas import tpu as pltpu
from jax.experimental.pallas import tpu_sc as plsc

D = 128
FF = 256
NG = 50
MAXR = 12.0
NGRAPHS = 16
T, B = 4096, 16
N = T * B
E = N
NGP = 64
MW = 16
BLK = 512

_STEP = MAXR / (NG - 1)
_COEFF = -0.5 / _STEP**2


def _dense_body(gsrc, gdst, msrc, mdst, eref,
                ew0, ew1, ew2, ebin, ewh1, ebh1, ewh2, ebh2, ewout, ebout, erw,
                fw0, fw1, fw2, fbin, fwh1, fbh1, fwh2, fbh2, fwout, fbout, frw,
                energy_ref, frows_ref):
    i = pl.program_id(0)
    a = gsrc[...]
    b = gdst[...]
    e = eref[...]

    lane = lax.broadcasted_iota(jnp.int32, (BLK, MW), 1)
    dm = msrc[...] - mdst[...]
    vec = jnp.where(lane < 3, dm, 0.0)
    d2 = jnp.sum(vec * vec, axis=1, keepdims=True)
    dist = jnp.sqrt(d2)
    vhat = vec / jnp.maximum(dist, 1e-12)
    batchf = jnp.sum(jnp.where(lane == 3, msrc[...], 0.0), axis=1, keepdims=True)

    off = lax.broadcasted_iota(jnp.int32, (BLK, NGP), 1).astype(jnp.float32) * _STEP
    rbf = jnp.exp(_COEFF * (dist - off) ** 2)

    def mlp(w0, w1, w2, bin_, wh1, bh1, wh2, bh2, wout, bout):
        h = jax.nn.gelu(
            jnp.dot(a, w0[...], preferred_element_type=jnp.float32)
            + jnp.dot(b, w1[...], preferred_element_type=jnp.float32)
            + jnp.dot(e, w2[...], preferred_element_type=jnp.float32)
            + bin_[...])
        h = h + jax.nn.gelu(jnp.dot(h, wh1[...], preferred_element_type=jnp.float32) + bh1[...])
        h = h + jax.nn.gelu(jnp.dot(h, wh2[...], preferred_element_type=jnp.float32) + bh2[...])
        return jnp.dot(h, wout[...], preferred_element_type=jnp.float32) + bout[...]

    m_e = mlp(ew0, ew1, ew2, ebin, ewh1, ebh1, ewh2, ebh2, ewout, ebout)
    m_f = mlp(fw0, fw1, fw2, fbin, fwh1, fbh1, fwh2, fbh2, fwout, fbout)

    rproj_e = jnp.dot(rbf, erw[...], preferred_element_type=jnp.float32)
    rproj_f = jnp.dot(rbf, frw[...], preferred_element_type=jnp.float32)

    prod_e = m_e * rproj_e
    f_scalar = jnp.sum(m_f * rproj_f, axis=1, keepdims=True)

    giota = lax.broadcasted_iota(jnp.int32, (BLK, NGRAPHS), 1).astype(jnp.float32)
    onehot = (batchf == giota).astype(jnp.float32)
    dg = lax.dot_general(onehot, prod_e, (((0,), (0,)), ((), ())),
                         preferred_element_type=jnp.float32)

    @pl.when(i == 0)
    def _():
        energy_ref[...] = jnp.zeros_like(energy_ref)

    energy_ref[...] += dg
    frows_ref[...] = f_scalar * vhat


def _dense_call(gsrc, gdst, msrc, mdst, nodes, wts):
    full = lambda shape: pl.BlockSpec(shape, lambda i: (0, 0))
    wspecs = []
    for w in wts:
        wspecs.append(full(w.shape))
    grid = (E // BLK,)
    return pl.pallas_call(
        _dense_body,
        grid=grid,
        in_specs=[
            pl.BlockSpec((BLK, D), lambda i: (i, 0)),
            pl.BlockSpec((BLK, D), lambda i: (i, 0)),
            pl.BlockSpec((BLK, MW), lambda i: (i, 0)),
            pl.BlockSpec((BLK, MW), lambda i: (i, 0)),
            pl.BlockSpec((BLK, D), lambda i: (i, 0)),
        ] + wspecs,
        out_specs=[
            pl.BlockSpec((NGRAPHS, D), lambda i: (0, 0)),
            pl.BlockSpec((BLK, MW), lambda i: (i, 0)),
        ],
        out_shape=[
            jax.ShapeDtypeStruct((NGRAPHS, D), jnp.float32),
            jax.ShapeDtypeStruct((E, MW), jnp.float32),
        ],
    )(gsrc, gdst, msrc, mdst, nodes, *wts)


def _gather_xla(nodes, meta, src, dst):
    return nodes[src], nodes[dst], meta[src], meta[dst]


def _scatter_xla(frows, src):
    return jax.ops.segment_sum(frows[:, :3], src, num_segments=N)


def kernel(x, pos, batch, edge_index, padded_node_mask, padded_edge_mask,
           eW_in, eb_in, eW_h1, eb_h1, eW_h2, eb_h2, eW_out, eb_out, e_rbf_W, e_out_W,
           fW_in, fb_in, fW_h1, fb_h1, fW_h2, fb_h2, fW_out, fb_out, f_rbf_W, f_out_W):
    nodes = jnp.transpose(x, (1, 0, 2)).reshape(N, D)
    meta = jnp.concatenate(
        [pos, batch.astype(jnp.float32)[:, None],
         jnp.zeros((N, MW - 4), jnp.float32)], axis=1)
    src = edge_index[0]
    dst = edge_index[1]

    erw = jnp.zeros((NGP, D), jnp.float32).at[:NG].set(e_rbf_W * e_out_W[:, 0][None, :])
    frw = jnp.zeros((NGP, D), jnp.float32).at[:NG].set(f_rbf_W * f_out_W[:, 0][None, :])

    wts = (eW_in[:D], eW_in[D:2 * D], eW_in[2 * D:], eb_in.reshape(1, FF),
           eW_h1, eb_h1.reshape(1, FF), eW_h2, eb_h2.reshape(1, FF),
           eW_out, eb_out.reshape(1, D), erw,
           fW_in[:D], fW_in[D:2 * D], fW_in[2 * D:], fb_in.reshape(1, FF),
           fW_h1, fb_h1.reshape(1, FF), fW_h2, fb_h2.reshape(1, FF),
           fW_out, fb_out.reshape(1, D), frw)

    gsrc, gdst, msrc, mdst = _gather_xla(nodes, meta, src, dst)
    energy_mat, frows = _dense_call(gsrc, gdst, msrc, mdst, nodes, wts)
    energy = jnp.sum(energy_mat, axis=1, keepdims=True)
    forces = _scatter_xla(frows, src)
    return (energy, forces)
